# baseline (device time: 45917 ns/iter reference)
import jax
import jax.numpy as jnp
from jax import lax
from jax.experimental import pallas as pl
from jax.experimental.pallas import tpu as pltpu

N_DEV = 4


def kernel(x, w_mat):
    m_per, k = x.shape
    _, n_per = w_mat.shape

    def body(x_ref, w_ref, out_ref, comm_ref, send_sems, recv_sems):
        my_pos = lax.axis_index("i")
        left = (my_pos - 1) % N_DEV
        right = (my_pos + 1) % N_DEV

        barrier_sem = pltpu.get_barrier_semaphore()
        for nbr in [left, right]:
            pl.semaphore_signal(
                barrier_sem, inc=1,
                device_id=(nbr,), device_id_type=pl.DeviceIdType.MESH,
            )
        pl.semaphore_wait(barrier_sem, 2)

        comm_ref[0] = x_ref[...]

        def compute_chunk(origin, chunk):
            y = jnp.dot(chunk, w_ref[...], preferred_element_type=jnp.float32)
            y = y * jax.nn.sigmoid(y)
            out_ref[pl.ds(origin * m_per, m_per), :] = y

        for h in range(N_DEV - 1):
            send_slot = h % 2
            recv_slot = (h + 1) % 2
            rdma = pltpu.make_async_remote_copy(
                src_ref=comm_ref.at[send_slot],
                dst_ref=comm_ref.at[recv_slot],
                send_sem=send_sems.at[send_slot],
                recv_sem=recv_sems.at[recv_slot],
                device_id=(right,),
                device_id_type=pl.DeviceIdType.MESH,
            )
            rdma.start()
            compute_chunk((my_pos - h) % N_DEV, comm_ref[send_slot])
            rdma.wait()
        compute_chunk((my_pos - (N_DEV - 1)) % N_DEV, comm_ref[(N_DEV - 1) % 2])

    return pl.pallas_call(
        body,
        out_shape=jax.ShapeDtypeStruct((N_DEV * m_per, n_per), jnp.float32),
        in_specs=[
            pl.BlockSpec(memory_space=pltpu.VMEM),
            pl.BlockSpec(memory_space=pltpu.VMEM),
        ],
        out_specs=pl.BlockSpec(memory_space=pltpu.VMEM),
        scratch_shapes=[
            pltpu.VMEM((2, m_per, k), jnp.float32),
            pltpu.SemaphoreType.DMA((2,)),
            pltpu.SemaphoreType.DMA((2,)),
        ],
        compiler_params=pltpu.CompilerParams(collective_id=0),
    )(x, w_mat)


# device time: 27452 ns/iter; 1.6726x vs baseline; 1.6726x over previous
import jax
import jax.numpy as jnp
from jax import lax
from jax.experimental import pallas as pl
from jax.experimental.pallas import tpu as pltpu

N_DEV = 4


def kernel(x, w_mat):
    m_per, k = x.shape
    _, n_per = w_mat.shape
    half = m_per // 2

    def body(x_ref, w_ref, out_ref, bl_ref, br_ref, bf_ref, send_sems, recv_sems):
        my_pos = lax.axis_index("i")
        left = (my_pos - 1) % N_DEV
        right = (my_pos + 1) % N_DEV

        barrier_sem = pltpu.get_barrier_semaphore()
        for nbr in [left, right]:
            pl.semaphore_signal(
                barrier_sem, inc=1,
                device_id=(nbr,), device_id_type=pl.DeviceIdType.MESH,
            )
        pl.semaphore_wait(barrier_sem, 2)

        s_r = pltpu.make_async_remote_copy(
            src_ref=x_ref, dst_ref=bl_ref,
            send_sem=send_sems.at[0], recv_sem=recv_sems.at[0],
            device_id=(right,), device_id_type=pl.DeviceIdType.MESH,
        )
        s_l = pltpu.make_async_remote_copy(
            src_ref=x_ref, dst_ref=br_ref,
            send_sem=send_sems.at[1], recv_sem=recv_sems.at[1],
            device_id=(left,), device_id_type=pl.DeviceIdType.MESH,
        )
        s_r.start()
        s_l.start()

        def compute(origin, chunk):
            y = jnp.dot(chunk, w_ref[...], preferred_element_type=jnp.float32)
            out_ref[pl.ds(origin * m_per, m_per), :] = y * jax.nn.sigmoid(y)

        compute(my_pos, x_ref[...])

        s_r.wait_recv()
        f_r = pltpu.make_async_remote_copy(
            src_ref=bl_ref.at[pl.ds(0, half)],
            dst_ref=bf_ref.at[pl.ds(0, half)],
            send_sem=send_sems.at[2], recv_sem=recv_sems.at[2],
            device_id=(right,), device_id_type=pl.DeviceIdType.MESH,
        )
        f_r.start()
        compute((my_pos - 1) % N_DEV, bl_ref[...])

        s_l.wait_recv()
        f_l = pltpu.make_async_remote_copy(
            src_ref=br_ref.at[pl.ds(half, half)],
            dst_ref=bf_ref.at[pl.ds(half, half)],
            send_sem=send_sems.at[3], recv_sem=recv_sems.at[3],
            device_id=(left,), device_id_type=pl.DeviceIdType.MESH,
        )
        f_l.start()
        compute((my_pos + 1) % N_DEV, br_ref[...])

        f_r.wait_recv()
        f_l.wait_recv()
        compute((my_pos + 2) % N_DEV, bf_ref[...])

        s_r.wait_send()
        s_l.wait_send()
        f_r.wait_send()
        f_l.wait_send()

    return pl.pallas_call(
        body,
        out_shape=jax.ShapeDtypeStruct((N_DEV * m_per, n_per), jnp.float32),
        in_specs=[
            pl.BlockSpec(memory_space=pltpu.VMEM),
            pl.BlockSpec(memory_space=pltpu.VMEM),
        ],
        out_specs=pl.BlockSpec(memory_space=pltpu.VMEM),
        scratch_shapes=[
            pltpu.VMEM((m_per, k), jnp.float32),
            pltpu.VMEM((m_per, k), jnp.float32),
            pltpu.VMEM((m_per, k), jnp.float32),
            pltpu.SemaphoreType.DMA((4,)),
            pltpu.SemaphoreType.DMA((4,)),
        ],
        compiler_params=pltpu.CompilerParams(collective_id=0),
    )(x, w_mat)


# device time: 26679 ns/iter; 1.7211x vs baseline; 1.0290x over previous
import jax
import jax.numpy as jnp
from jax import lax
from jax.experimental import pallas as pl
from jax.experimental.pallas import tpu as pltpu

N_DEV = 4


def kernel(x, w_mat):
    m_per, k = x.shape
    _, n_per = w_mat.shape
    half = m_per // 2

    def body(x_ref, w_ref, out_ref, bl_ref, br_ref, bf_ref, send_sems, recv_sems):
        my_pos = lax.axis_index("i")
        left = (my_pos - 1) % N_DEV
        right = (my_pos + 1) % N_DEV

        barrier_sem = pltpu.get_barrier_semaphore()
        for nbr in [left, right]:
            pl.semaphore_signal(
                barrier_sem, inc=1,
                device_id=(nbr,), device_id_type=pl.DeviceIdType.MESH,
            )
        pl.semaphore_wait(barrier_sem, 2)

        s_r = pltpu.make_async_remote_copy(
            src_ref=x_ref, dst_ref=bl_ref,
            send_sem=send_sems.at[0], recv_sem=recv_sems.at[0],
            device_id=(right,), device_id_type=pl.DeviceIdType.MESH,
        )
        s_l = pltpu.make_async_remote_copy(
            src_ref=x_ref, dst_ref=br_ref,
            send_sem=send_sems.at[1], recv_sem=recv_sems.at[1],
            device_id=(left,), device_id_type=pl.DeviceIdType.MESH,
        )
        s_r.start()
        s_l.start()

        def compute(origin, chunk):
            y = jnp.dot(chunk, w_ref[...], preferred_element_type=jnp.float32)
            out_ref[pl.ds(origin * m_per, m_per), :] = y * jax.nn.sigmoid(y)

        compute(my_pos, x_ref[...])

        s_r.wait_recv()
        f_r = pltpu.make_async_remote_copy(
            src_ref=bl_ref.at[pl.ds(0, half)],
            dst_ref=bf_ref.at[pl.ds(0, half)],
            send_sem=send_sems.at[2], recv_sem=recv_sems.at[2],
            device_id=(right,), device_id_type=pl.DeviceIdType.MESH,
        )
        f_r.start()

        s_l.wait_recv()
        f_l = pltpu.make_async_remote_copy(
            src_ref=br_ref.at[pl.ds(half, half)],
            dst_ref=bf_ref.at[pl.ds(half, half)],
            send_sem=send_sems.at[3], recv_sem=recv_sems.at[3],
            device_id=(left,), device_id_type=pl.DeviceIdType.MESH,
        )
        f_l.start()
        compute((my_pos - 1) % N_DEV, bl_ref[...])
        compute((my_pos + 1) % N_DEV, br_ref[...])

        f_r.wait_recv()
        f_l.wait_recv()
        compute((my_pos + 2) % N_DEV, bf_ref[...])

        s_r.wait_send()
        s_l.wait_send()
        f_r.wait_send()
        f_l.wait_send()

    return pl.pallas_call(
        body,
        out_shape=jax.ShapeDtypeStruct((N_DEV * m_per, n_per), jnp.float32),
        in_specs=[
            pl.BlockSpec(memory_space=pltpu.VMEM),
            pl.BlockSpec(memory_space=pltpu.VMEM),
        ],
        out_specs=pl.BlockSpec(memory_space=pltpu.VMEM),
        scratch_shapes=[
            pltpu.VMEM((m_per, k), jnp.float32),
            pltpu.VMEM((m_per, k), jnp.float32),
            pltpu.VMEM((m_per, k), jnp.float32),
            pltpu.SemaphoreType.DMA((4,)),
            pltpu.SemaphoreType.DMA((4,)),
        ],
        compiler_params=pltpu.CompilerParams(collective_id=0),
    )(x, w_mat)


# device time: 25780 ns/iter; 1.7811x vs baseline; 1.0349x over previous
import jax
import jax.numpy as jnp
from jax import lax
from jax.experimental import pallas as pl
from jax.experimental.pallas import tpu as pltpu

N_DEV = 4


def kernel(x, w_mat):
    m_per, k = x.shape
    _, n_per = w_mat.shape
    half = m_per // 2
    lo = pl.ds(0, half)
    hi = pl.ds(half, half)

    def body(x_ref, w_ref, out_ref, bl_ref, br_ref, bf_ref, send_sems, recv_sems):
        my_pos = lax.axis_index("i")
        left = (my_pos - 1) % N_DEV
        right = (my_pos + 1) % N_DEV

        barrier_sem = pltpu.get_barrier_semaphore()
        for nbr in [left, right]:
            pl.semaphore_signal(
                barrier_sem, inc=1,
                device_id=(nbr,), device_id_type=pl.DeviceIdType.MESH,
            )
        pl.semaphore_wait(barrier_sem, 2)

        def rdma(src, dst, i, dev):
            return pltpu.make_async_remote_copy(
                src_ref=src, dst_ref=dst,
                send_sem=send_sems.at[i], recv_sem=recv_sems.at[i],
                device_id=(dev,), device_id_type=pl.DeviceIdType.MESH,
            )

        s_r_a = rdma(x_ref.at[lo], bl_ref.at[lo], 0, right)
        s_r_b = rdma(x_ref.at[hi], bl_ref.at[hi], 1, right)
        s_l_a = rdma(x_ref.at[hi], br_ref.at[hi], 2, left)
        s_l_b = rdma(x_ref.at[lo], br_ref.at[lo], 3, left)
        s_r_a.start()
        s_r_b.start()
        s_l_a.start()
        s_l_b.start()

        def compute(origin, chunk):
            y = jnp.dot(chunk, w_ref[...], preferred_element_type=jnp.float32)
            out_ref[pl.ds(origin * m_per, m_per), :] = y * jax.nn.sigmoid(y)

        compute(my_pos, x_ref[...])

        s_r_a.wait_recv()
        f_r = rdma(bl_ref.at[lo], bf_ref.at[lo], 4, right)
        f_r.start()
        s_l_a.wait_recv()
        f_l = rdma(br_ref.at[hi], bf_ref.at[hi], 5, left)
        f_l.start()

        s_r_b.wait_recv()
        compute((my_pos - 1) % N_DEV, bl_ref[...])
        s_l_b.wait_recv()
        compute((my_pos + 1) % N_DEV, br_ref[...])

        f_r.wait_recv()
        f_l.wait_recv()
        compute((my_pos + 2) % N_DEV, bf_ref[...])

        for d in (s_r_a, s_r_b, s_l_a, s_l_b, f_r, f_l):
            d.wait_send()

    return pl.pallas_call(
        body,
        out_shape=jax.ShapeDtypeStruct((N_DEV * m_per, n_per), jnp.float32),
        in_specs=[
            pl.BlockSpec(memory_space=pltpu.VMEM),
            pl.BlockSpec(memory_space=pltpu.VMEM),
        ],
        out_specs=pl.BlockSpec(memory_space=pltpu.VMEM),
        scratch_shapes=[
            pltpu.VMEM((m_per, k), jnp.float32),
            pltpu.VMEM((m_per, k), jnp.float32),
            pltpu.VMEM((m_per, k), jnp.float32),
            pltpu.SemaphoreType.DMA((6,)),
            pltpu.SemaphoreType.DMA((6,)),
        ],
        compiler_params=pltpu.CompilerParams(collective_id=0),
    )(x, w_mat)
